# baseline (speedup 1.0000x reference)
import jax
import jax.numpy as jnp
from jax.experimental import pallas as pl
from jax.experimental.pallas import tpu as pltpu

B, N, C = 8, 20000, 80
NBUF = 4

_CHUNKS = (
    [(0, 0, 8), (0, 8, 16), (0, 24, 24), (0, 48, 32)]
    + [(b, 0, 80) for b in range(1, 7)]
    + [(7, 0, 32), (7, 32, 24), (7, 56, 16), (7, 72, 8)]
)
NCHUNK = len(_CHUNKS)
MAXR = 80


def _manual_kernel(lt_ref, ct_ref, rt_ref, pt_ref, lo_ref, bo_ref,
                   lbuf, obuf, cbuf, rbuf, pbuf, bbuf,
                   sin, sout, saux, sbox):
    cp_c = pltpu.make_async_copy(ct_ref, cbuf, saux.at[0])
    cp_r = pltpu.make_async_copy(rt_ref, rbuf, saux.at[1])
    cp_p = pltpu.make_async_copy(pt_ref, pbuf, saux.at[2])
    cp_c.start()
    cp_r.start()
    cp_p.start()

    def in_copy(i, slot):
        b, r0, nr = _CHUNKS[i]
        return pltpu.make_async_copy(
            lt_ref.at[b, pl.ds(r0, nr)], lbuf.at[slot, pl.ds(0, nr)],
            sin.at[slot])

    def out_copy(i, slot):
        b, r0, nr = _CHUNKS[i]
        return pltpu.make_async_copy(
            obuf.at[slot, pl.ds(0, nr)], lo_ref.at[b, pl.ds(r0, nr)],
            sout.at[slot])

    for i in range(NBUF):
        in_copy(i, i).start()

    cp_r.wait()
    cp_p.wait()
    r = rbuf[...]
    px = pbuf[0:1, :][None]
    py = pbuf[1:2, :][None]
    row = jax.lax.broadcasted_iota(jnp.int32, r.shape, 1)
    sign = jnp.where(row >= 2, 1.0, -1.0).astype(jnp.float32)
    pts4 = jnp.where(row % 2 == 0, px, py)
    bbuf[...] = jnp.clip(pts4 + sign * r, 0.0, 1.0)
    box_dma = pltpu.make_async_copy(bbuf, bo_ref, sbox)
    box_dma.start()
    cp_c.wait()

    for i in range(NCHUNK):
        slot = i % NBUF
        b, _, nr = _CHUNKS[i]
        in_copy(i, slot).wait()
        if i >= NBUF:
            out_copy(i - NBUF, slot).wait()
        c = cbuf[b, 0]
        obuf[slot, pl.ds(0, nr)] = jnp.sqrt(lbuf[slot, pl.ds(0, nr)] * c[None, :])
        out_copy(i, slot).start()
        if i + NBUF < NCHUNK:
            in_copy(i + NBUF, slot).start()

    for i in range(NCHUNK - NBUF, NCHUNK):
        out_copy(i, i % NBUF).wait()
    box_dma.wait()


def kernel(logits, regress, points, centerness):
    lt = jnp.transpose(logits, (0, 2, 1))
    rt = jnp.transpose(regress, (0, 2, 1))
    pt = jnp.transpose(points, (1, 0))
    ct = jnp.transpose(centerness, (0, 2, 1))

    hbm = pl.BlockSpec(memory_space=pltpu.HBM)
    out = pl.pallas_call(
        _manual_kernel,
        in_specs=[hbm, hbm, hbm, hbm],
        out_specs=[hbm, hbm],
        out_shape=[
            jax.ShapeDtypeStruct((B, C, N), jnp.float32),
            jax.ShapeDtypeStruct((B, 4, N), jnp.float32),
        ],
        scratch_shapes=[
            pltpu.VMEM((NBUF, MAXR, N), jnp.float32),
            pltpu.VMEM((NBUF, MAXR, N), jnp.float32),
            pltpu.VMEM((B, 1, N), jnp.float32),
            pltpu.VMEM((B, 4, N), jnp.float32),
            pltpu.VMEM((2, N), jnp.float32),
            pltpu.VMEM((B, 4, N), jnp.float32),
            pltpu.SemaphoreType.DMA((NBUF,)),
            pltpu.SemaphoreType.DMA((NBUF,)),
            pltpu.SemaphoreType.DMA((3,)),
            pltpu.SemaphoreType.DMA,
        ],
    )(lt, ct, rt, pt)
    return (jnp.transpose(out[0], (0, 2, 1)), jnp.transpose(out[1], (0, 2, 1)))


# speedup vs baseline: 1.0326x; 1.0326x over previous
import jax
import jax.numpy as jnp
from jax.experimental import pallas as pl
from jax.experimental.pallas import tpu as pltpu

B, N, C = 8, 20000, 80
NBUF = 4

_CHUNKS = (
    [(0, 0, 8), (0, 8, 24), (0, 32, 48)]
    + [(b, 0, 80) for b in range(1, 7)]
    + [(7, 0, 48), (7, 48, 24), (7, 72, 8)]
)
NCHUNK = len(_CHUNKS)
MAXR = 80


def _manual_kernel(lt_ref, ct_ref, rt_ref, pt_ref, lo_ref, bo_ref,
                   lbuf, obuf, cbuf, rbuf, pbuf, bbuf,
                   sin, sout, saux, sbox):
    cp_c = pltpu.make_async_copy(ct_ref, cbuf, saux.at[0])
    cp_r = pltpu.make_async_copy(rt_ref, rbuf, saux.at[1])
    cp_p = pltpu.make_async_copy(pt_ref, pbuf, saux.at[2])
    cp_c.start()
    cp_r.start()
    cp_p.start()

    def in_copy(i, slot):
        b, r0, nr = _CHUNKS[i]
        return pltpu.make_async_copy(
            lt_ref.at[b, pl.ds(r0, nr)], lbuf.at[slot, pl.ds(0, nr)],
            sin.at[slot])

    def out_copy(i, slot):
        b, r0, nr = _CHUNKS[i]
        return pltpu.make_async_copy(
            obuf.at[slot, pl.ds(0, nr)], lo_ref.at[b, pl.ds(r0, nr)],
            sout.at[slot])

    for i in range(NBUF):
        in_copy(i, i).start()

    cp_r.wait()
    cp_p.wait()
    r = rbuf[...]
    px = pbuf[0:1, :][None]
    py = pbuf[1:2, :][None]
    row = jax.lax.broadcasted_iota(jnp.int32, r.shape, 1)
    sign = jnp.where(row >= 2, 1.0, -1.0).astype(jnp.float32)
    pts4 = jnp.where(row % 2 == 0, px, py)
    bbuf[...] = jnp.clip(pts4 + sign * r, 0.0, 1.0)
    box_dma = pltpu.make_async_copy(bbuf, bo_ref, sbox)
    box_dma.start()
    cp_c.wait()

    for i in range(NCHUNK):
        slot = i % NBUF
        b, _, nr = _CHUNKS[i]
        in_copy(i, slot).wait()
        if i >= NBUF:
            out_copy(i - NBUF, slot).wait()
        c = cbuf[b, 0]
        obuf[slot, pl.ds(0, nr)] = jnp.sqrt(lbuf[slot, pl.ds(0, nr)] * c[None, :])
        out_copy(i, slot).start()
        if i + NBUF < NCHUNK:
            in_copy(i + NBUF, slot).start()

    for i in range(NCHUNK - NBUF, NCHUNK):
        out_copy(i, i % NBUF).wait()
    box_dma.wait()


def kernel(logits, regress, points, centerness):
    lt = jnp.transpose(logits, (0, 2, 1))
    rt = jnp.transpose(regress, (0, 2, 1))
    pt = jnp.transpose(points, (1, 0))
    ct = jnp.transpose(centerness, (0, 2, 1))

    hbm = pl.BlockSpec(memory_space=pltpu.HBM)
    out = pl.pallas_call(
        _manual_kernel,
        in_specs=[hbm, hbm, hbm, hbm],
        out_specs=[hbm, hbm],
        out_shape=[
            jax.ShapeDtypeStruct((B, C, N), jnp.float32),
            jax.ShapeDtypeStruct((B, 4, N), jnp.float32),
        ],
        scratch_shapes=[
            pltpu.VMEM((NBUF, MAXR, N), jnp.float32),
            pltpu.VMEM((NBUF, MAXR, N), jnp.float32),
            pltpu.VMEM((B, 1, N), jnp.float32),
            pltpu.VMEM((B, 4, N), jnp.float32),
            pltpu.VMEM((2, N), jnp.float32),
            pltpu.VMEM((B, 4, N), jnp.float32),
            pltpu.SemaphoreType.DMA((NBUF,)),
            pltpu.SemaphoreType.DMA((NBUF,)),
            pltpu.SemaphoreType.DMA((3,)),
            pltpu.SemaphoreType.DMA,
        ],
    )(lt, ct, rt, pt)
    return (jnp.transpose(out[0], (0, 2, 1)), jnp.transpose(out[1], (0, 2, 1)))
